# baseline (speedup 1.0000x reference)
import functools

import jax
import jax.numpy as jnp
from jax import lax
from jax.experimental import pallas as pl
from jax.experimental.pallas import tpu as pltpu
from jax.experimental.pallas import tpu_sc as plsc

N = 10000
NP = 10240
D = 128
D2 = 64
NC = 2
NS = 16
L = 16
W = 80

_MESH = plsc.VectorSubcoreMesh(
    core_axis_name="c", subcore_axis_name="s", num_cores=NC, num_subcores=NS
)


def _zero_fill2d(buf, rows, cols):
    z = jnp.zeros((L,), jnp.float32)
    per_row = cols // L

    @pl.loop(0, rows * per_row)
    def _(i):
        buf[i // per_row, pl.ds((i % per_row) * L, L)] = z


CHUNK = 25


def _agg_body(nwin, x_hbm, src_hbm, dst_hbm, out_hbm,
              agg_sh, gbuf, srcv, dstv, sem):
    cid = lax.axis_index("c")
    sid = lax.axis_index("s")
    wid = cid * NS + sid
    rows_per_tile = NP // NS

    _zero_fill2d(gbuf.at[0], W, D)
    for k in range(rows_per_tile // W):
        pltpu.sync_copy(gbuf.at[0],
                        agg_sh.at[pl.ds(sid * rows_per_tile + k * W, W)])
    plsc.subcore_barrier()

    def _gather(g, b):
        return pltpu.async_copy(x_hbm.at[srcv.at[g]], gbuf.at[b], sem.at[b])

    for chunk in range(nwin // CHUNK):
        pltpu.sync_copy(src_hbm.at[wid, chunk], srcv)
        pltpu.sync_copy(dst_hbm.at[wid, chunk], dstv)
        _gather(0, 0)

        @pl.loop(0, CHUNK)
        def _(g):
            b = lax.rem(g, 2)

            @pl.when(g + 1 < CHUNK)
            def _():
                _gather(g + 1, 1 - b)

            pltpu.make_async_copy(x_hbm.at[srcv.at[g]], gbuf.at[b],
                                  sem.at[b]).wait()
            pltpu.sync_copy(gbuf.at[b], agg_sh.at[dstv.at[g]], add=True)

    plsc.subcore_barrier()
    pltpu.sync_copy(agg_sh.at[pl.ds(sid * rows_per_tile, rows_per_tile)],
                    out_hbm.at[cid, pl.ds(sid * rows_per_tile, rows_per_tile)])


def _agg(x, src3d, dst3d):
    nwin = src3d.shape[1] * src3d.shape[2]
    body = functools.partial(_agg_body, nwin)
    return pl.kernel(
        body,
        out_type=jax.ShapeDtypeStruct((NC, NP, D), jnp.float32),
        mesh=_MESH,
        compiler_params=pltpu.CompilerParams(needs_layout_passes=False),
        scratch_types=[
            pltpu.VMEM_SHARED((NP, D), jnp.float32),
            pltpu.VMEM((2, W, D), jnp.float32),
            pltpu.VMEM((CHUNK, W), jnp.int32),
            pltpu.VMEM((CHUNK, W), jnp.int32),
            pltpu.SemaphoreType.DMA((2,)),
        ],
    )(x, src3d, dst3d)


def _mlp_kernel(x_ref, agg_ref, wa_ref, ba_ref, wb_ref, bb_ref, out_ref):
    h = x_ref[...] + agg_ref[0] + agg_ref[1]
    w = jnp.dot(wa_ref[...], wb_ref[...], preferred_element_type=jnp.float32)
    b = jnp.dot(ba_ref[...], wb_ref[...], preferred_element_type=jnp.float32) + bb_ref[...]
    out_ref[...] = jnp.dot(h, w, preferred_element_type=jnp.float32) + b


def _mlp(x, agg, wa, ba, wb, bb):
    n, d = x.shape
    dout = wb.shape[1]
    blk = 1024
    return pl.pallas_call(
        _mlp_kernel,
        grid=(n // blk,),
        in_specs=[
            pl.BlockSpec((blk, d), lambda i: (i, 0)),
            pl.BlockSpec((NC, blk, d), lambda i: (0, i, 0)),
            pl.BlockSpec(wa.shape, lambda i: (0, 0)),
            pl.BlockSpec((1, ba.shape[0]), lambda i: (0, 0)),
            pl.BlockSpec(wb.shape, lambda i: (0, 0)),
            pl.BlockSpec((1, dout), lambda i: (0, 0)),
        ],
        out_specs=pl.BlockSpec((blk, dout), lambda i: (i, 0)),
        out_shape=jax.ShapeDtypeStruct((n, dout), jnp.float32),
    )(x, agg, wa, ba[None, :], wb, bb[None, :])


SCHUNK = 25
SW = 40


def _score_body(nwin, h_hbm, ii_hbm, jj_hbm, out_hbm,
                h_sh, iv, jv, xi, xj, outv, sem):
    cid = lax.axis_index("c")
    sid = lax.axis_index("s")
    wid = cid * NS + sid
    rows_per_tile = NP // NS

    pltpu.sync_copy(h_hbm.at[pl.ds(sid * rows_per_tile, rows_per_tile)],
                    h_sh.at[pl.ds(sid * rows_per_tile, rows_per_tile)])
    plsc.subcore_barrier()

    lanes = lax.iota(jnp.int32, L)

    def _gather(g, b):
        pltpu.async_copy(h_hbm.at[iv.at[g]], xi.at[b], sem.at[2 * b])
        pltpu.async_copy(h_sh.at[jv.at[g]], xj.at[b], sem.at[2 * b + 1])

    def _wait(g, b):
        pltpu.make_async_copy(h_hbm.at[iv.at[g]], xi.at[b], sem.at[2 * b]).wait()
        pltpu.make_async_copy(h_sh.at[jv.at[g]], xj.at[b], sem.at[2 * b + 1]).wait()

    @pl.loop(0, nwin)
    def _(gw):
        chunk = gw // SCHUNK
        g = lax.rem(gw, SCHUNK)
        b = lax.rem(gw, 2)

        @pl.when(g == 0)
        def _():
            pltpu.sync_copy(ii_hbm.at[wid, chunk], iv)
            pltpu.sync_copy(jj_hbm.at[wid, chunk], jv)
            _gather(0, b)

        _wait(g, b)

        @pl.when(jnp.logical_and(g + 1 < SCHUNK, gw + 1 < nwin))
        def _():
            _gather(g + 1, 1 - b)

        for grp in range(SW // L):
            out_vec = jnp.zeros((L,), jnp.float32)
            for e16 in range(L):
                e = grp * L + e16
                p = xi[b, e, pl.ds(0, L)] * xj[b, e, pl.ds(0, L)]
                for k in range(1, D2 // L):
                    p = p + (xi[b, e, pl.ds(k * L, L)]
                             * xj[b, e, pl.ds(k * L, L)])
                out_vec = jnp.where(lanes == e16, jnp.sum(p), out_vec)
            outv[pl.ds(grp * L, L)] = out_vec
        pltpu.sync_copy(outv, out_hbm.at[pl.ds(wid * nwin * SW + gw * SW, SW)])


def _score(h2, ii3d, jj3d):
    nwin = ii3d.shape[1] * ii3d.shape[2]
    e2 = NC * NS * nwin * SW
    body = functools.partial(_score_body, nwin)
    return pl.kernel(
        body,
        out_type=jax.ShapeDtypeStruct((e2,), jnp.float32),
        mesh=_MESH,
        compiler_params=pltpu.CompilerParams(needs_layout_passes=False),
        scratch_types=[
            pltpu.VMEM_SHARED((NP, D), jnp.float32),
            pltpu.VMEM((SCHUNK, SW), jnp.int32),
            pltpu.VMEM((SCHUNK, SW), jnp.int32),
            pltpu.VMEM((2, SW, D), jnp.float32),
            pltpu.VMEM((2, SW, D), jnp.float32),
            pltpu.VMEM((SW,), jnp.float32),
            pltpu.SemaphoreType.DMA((4,)),
        ],
    )(h2, ii3d, jj3d)


def kernel(x, W1a, b1a, W1b, b1b, W2a, b2a, W2b, b2b,
           train_pos_edge_index, pos_edge_index, neg_edge_index):
    nt = NC * NS
    xp = jnp.pad(x, ((0, NP - N), (0, 0)))
    src3d = train_pos_edge_index[0].reshape(nt, -1, CHUNK, W)
    dst3d = train_pos_edge_index[1].reshape(nt, -1, CHUNK, W)

    agg1 = _agg(xp, src3d, dst3d)
    h = _mlp(xp, agg1, W1a, b1a, W1b, b1b)
    agg2 = _agg(h, src3d, dst3d)
    W2bp = jnp.pad(W2b, ((0, 0), (0, D - D2)))
    b2bp = jnp.pad(b2b, (0, D - D2))
    h2 = _mlp(h, agg2, W2a, b2a, W2bp, b2bp)

    ii3d = jnp.concatenate([pos_edge_index[0], neg_edge_index[0]]).reshape(nt, -1, SCHUNK, SW)
    jj3d = jnp.concatenate([pos_edge_index[1], neg_edge_index[1]]).reshape(nt, -1, SCHUNK, SW)
    return _score(h2, ii3d, jj3d)


# speedup vs baseline: 1.1816x; 1.1816x over previous
import functools

import jax
import jax.numpy as jnp
from jax import lax
from jax.experimental import pallas as pl
from jax.experimental.pallas import tpu as pltpu
from jax.experimental.pallas import tpu_sc as plsc

N = 10000
NP = 10240
D = 128
D2 = 64
NC = 2
NS = 16
L = 16
W = 80
AW = 125
ACHUNK = 20

_MESH = plsc.VectorSubcoreMesh(
    core_axis_name="c", subcore_axis_name="s", num_cores=NC, num_subcores=NS
)


def _zero_fill2d(buf, rows, cols):
    z = jnp.zeros((L,), jnp.float32)
    per_row = cols // L

    @pl.loop(0, rows * per_row)
    def _(i):
        buf[i // per_row, pl.ds((i % per_row) * L, L)] = z


def _agg_body(nwin, x_hbm, src_hbm, dst_hbm, out_hbm,
              agg_sh, gbuf, srcv, dstv, sem):
    cid = lax.axis_index("c")
    sid = lax.axis_index("s")
    wid = cid * NS + sid
    rows_per_tile = NP // NS

    _zero_fill2d(gbuf.at[0], AW, D)
    for k in range(rows_per_tile // 125):
        pltpu.sync_copy(gbuf.at[0].at[pl.ds(0, 125)],
                        agg_sh.at[pl.ds(sid * rows_per_tile + k * 125, 125)])
    pltpu.sync_copy(
        gbuf.at[0].at[pl.ds(0, rows_per_tile % 125)],
        agg_sh.at[pl.ds(sid * rows_per_tile + 5 * 125, rows_per_tile % 125)])
    plsc.subcore_barrier()

    def _gather(g, b):
        return pltpu.async_copy(x_hbm.at[srcv.at[g]], gbuf.at[b], sem.at[b])

    for chunk in range(nwin // ACHUNK):
        pltpu.sync_copy(src_hbm.at[wid, chunk], srcv)
        pltpu.sync_copy(dst_hbm.at[wid, chunk], dstv)
        _gather(0, 0)

        @pl.loop(0, ACHUNK)
        def _(g):
            b = lax.rem(g, 2)

            @pl.when(g + 1 < ACHUNK)
            def _():
                _gather(g + 1, 1 - b)

            pltpu.make_async_copy(x_hbm.at[srcv.at[g]], gbuf.at[b],
                                  sem.at[b]).wait()
            pltpu.sync_copy(gbuf.at[b], agg_sh.at[dstv.at[g]], add=True)

    plsc.subcore_barrier()
    pltpu.sync_copy(agg_sh.at[pl.ds(sid * rows_per_tile, rows_per_tile)],
                    out_hbm.at[cid, pl.ds(sid * rows_per_tile, rows_per_tile)])


def _agg(x, src3d, dst3d):
    nwin = src3d.shape[1] * src3d.shape[2]
    body = functools.partial(_agg_body, nwin)
    return pl.kernel(
        body,
        out_type=jax.ShapeDtypeStruct((NC, NP, D), jnp.float32),
        mesh=_MESH,
        compiler_params=pltpu.CompilerParams(needs_layout_passes=False),
        scratch_types=[
            pltpu.VMEM_SHARED((NP, D), jnp.float32),
            pltpu.VMEM((2, AW, D), jnp.float32),
            pltpu.VMEM((ACHUNK, AW), jnp.int32),
            pltpu.VMEM((ACHUNK, AW), jnp.int32),
            pltpu.SemaphoreType.DMA((2,)),
        ],
    )(x, src3d, dst3d)


def _mlp_kernel(x_ref, agg_ref, wa_ref, ba_ref, wb_ref, bb_ref, out_ref):
    h = x_ref[...] + agg_ref[0] + agg_ref[1]
    w = jnp.dot(wa_ref[...], wb_ref[...], preferred_element_type=jnp.float32)
    b = jnp.dot(ba_ref[...], wb_ref[...], preferred_element_type=jnp.float32) + bb_ref[...]
    out_ref[...] = jnp.dot(h, w, preferred_element_type=jnp.float32) + b


def _mlp(x, agg, wa, ba, wb, bb):
    n, d = x.shape
    dout = wb.shape[1]
    blk = 1024
    return pl.pallas_call(
        _mlp_kernel,
        grid=(n // blk,),
        in_specs=[
            pl.BlockSpec((blk, d), lambda i: (i, 0)),
            pl.BlockSpec((NC, blk, d), lambda i: (0, i, 0)),
            pl.BlockSpec(wa.shape, lambda i: (0, 0)),
            pl.BlockSpec((1, ba.shape[0]), lambda i: (0, 0)),
            pl.BlockSpec(wb.shape, lambda i: (0, 0)),
            pl.BlockSpec((1, dout), lambda i: (0, 0)),
        ],
        out_specs=pl.BlockSpec((blk, dout), lambda i: (i, 0)),
        out_shape=jax.ShapeDtypeStruct((n, dout), jnp.float32),
    )(x, agg, wa, ba[None, :], wb, bb[None, :])


SCHUNK = 25


def _score_body(nwin, h_hbm, ii_hbm, jj_hbm, out_hbm,
                iv, jv, xi, xj, outv, sem):
    cid = lax.axis_index("c")
    sid = lax.axis_index("s")
    wid = cid * NS + sid

    lanes = lax.iota(jnp.int32, L)

    def _gather(g, b):
        pltpu.async_copy(h_hbm.at[iv.at[g]], xi.at[b], sem.at[2 * b])
        pltpu.async_copy(h_hbm.at[jv.at[g]], xj.at[b], sem.at[2 * b + 1])

    def _wait(g, b):
        pltpu.make_async_copy(h_hbm.at[iv.at[g]], xi.at[b], sem.at[2 * b]).wait()
        pltpu.make_async_copy(h_hbm.at[jv.at[g]], xj.at[b], sem.at[2 * b + 1]).wait()

    nwv = nwin
    @pl.loop(0, nwin)
    def _(gw):
        chunk = gw // SCHUNK
        g = lax.rem(gw, SCHUNK)
        b = lax.rem(gw, 2)

        @pl.when(g == 0)
        def _():
            pltpu.sync_copy(ii_hbm.at[wid, chunk], iv)
            pltpu.sync_copy(jj_hbm.at[wid, chunk], jv)
            _gather(0, b)

        _wait(g, b)

        @pl.when(jnp.logical_and(g + 1 < SCHUNK, gw + 1 < nwin))
        def _():
            _gather(g + 1, 1 - b)

        for grp in range(W // L):
            out_vec = jnp.zeros((L,), jnp.float32)
            for e16 in range(L):
                e = grp * L + e16
                p = xi[b, e, pl.ds(0, L)] * xj[b, e, pl.ds(0, L)]
                for k in range(1, D2 // L):
                    p = p + (xi[b, e, pl.ds(k * L, L)]
                             * xj[b, e, pl.ds(k * L, L)])
                out_vec = jnp.where(lanes == e16, jnp.sum(p), out_vec)
            outv[pl.ds(gw * W + grp * L, L)] = out_vec


def _score(h2, ii3d, jj3d):
    nwin = ii3d.shape[1] * ii3d.shape[2]
    e2 = NC * NS * nwin * W
    body = functools.partial(_score_body, nwin)
    return pl.kernel(
        body,
        out_type=jax.ShapeDtypeStruct((e2,), jnp.float32),
        mesh=_MESH,
        compiler_params=pltpu.CompilerParams(needs_layout_passes=False),
        scratch_types=[
            pltpu.VMEM((SCHUNK, W), jnp.int32),
            pltpu.VMEM((SCHUNK, W), jnp.int32),
            pltpu.VMEM((2, W, D), jnp.float32),
            pltpu.VMEM((2, W, D), jnp.float32),
            pltpu.VMEM((250 * W,), jnp.float32),
            pltpu.SemaphoreType.DMA((4,)),
        ],
    )(h2, ii3d, jj3d)


def kernel(x, W1a, b1a, W1b, b1b, W2a, b2a, W2b, b2b,
           train_pos_edge_index, pos_edge_index, neg_edge_index):
    nt = NC * NS
    xp = jnp.pad(x, ((0, NP - N), (0, 0)))
    src3d = train_pos_edge_index[0].reshape(nt, -1, ACHUNK, AW)
    dst3d = train_pos_edge_index[1].reshape(nt, -1, ACHUNK, AW)

    agg1 = _agg(xp, src3d, dst3d)
    h = _mlp(xp, agg1, W1a, b1a, W1b, b1b)
    agg2 = _agg(h, src3d, dst3d)
    W2bp = jnp.pad(W2b, ((0, 0), (0, D - D2)))
    b2bp = jnp.pad(b2b, (0, D - D2))
    h2 = _mlp(h, agg2, W2a, b2a, W2bp, b2bp)

    ii3d = jnp.concatenate([pos_edge_index[0], neg_edge_index[0]]).reshape(nt, -1, SCHUNK, W)
    jj3d = jnp.concatenate([pos_edge_index[1], neg_edge_index[1]]).reshape(nt, -1, SCHUNK, W)
    return _score(h2, ii3d, jj3d)


# speedup vs baseline: 1.2737x; 1.0780x over previous
import functools

import jax
import jax.numpy as jnp
from jax import lax
from jax.experimental import pallas as pl
from jax.experimental.pallas import tpu as pltpu
from jax.experimental.pallas import tpu_sc as plsc

N = 10000
NP = 10240
D = 128
D2 = 64
NC = 2
NS = 16
L = 16
W = 80
AW = 125
ACHUNK = 20

_MESH = plsc.VectorSubcoreMesh(
    core_axis_name="c", subcore_axis_name="s", num_cores=NC, num_subcores=NS
)


def _zero_fill2d(buf, rows, cols):
    z = jnp.zeros((L,), jnp.float32)
    per_row = cols // L

    @pl.loop(0, rows * per_row)
    def _(i):
        buf[i // per_row, pl.ds((i % per_row) * L, L)] = z


def _agg_body(nwin, x_hbm, src_hbm, dst_hbm, out_hbm,
              agg_sh, gbuf, srcv, dstv, sem):
    cid = lax.axis_index("c")
    sid = lax.axis_index("s")
    wid = cid * NS + sid
    rows_per_tile = NP // NS

    _zero_fill2d(gbuf.at[0], AW, D)
    for k in range(rows_per_tile // 125):
        pltpu.sync_copy(gbuf.at[0].at[pl.ds(0, 125)],
                        agg_sh.at[pl.ds(sid * rows_per_tile + k * 125, 125)])
    pltpu.sync_copy(
        gbuf.at[0].at[pl.ds(0, rows_per_tile % 125)],
        agg_sh.at[pl.ds(sid * rows_per_tile + 5 * 125, rows_per_tile % 125)])
    plsc.subcore_barrier()

    def _gather(g, b):
        return pltpu.async_copy(x_hbm.at[srcv.at[g]], gbuf.at[b], sem.at[b])

    for chunk in range(nwin // ACHUNK):
        pltpu.sync_copy(src_hbm.at[wid, chunk], srcv)
        pltpu.sync_copy(dst_hbm.at[wid, chunk], dstv)
        _gather(0, 0)

        @pl.loop(0, ACHUNK)
        def _(g):
            b = lax.rem(g, 2)

            @pl.when(g + 1 < ACHUNK)
            def _():
                _gather(g + 1, 1 - b)

            pltpu.make_async_copy(x_hbm.at[srcv.at[g]], gbuf.at[b],
                                  sem.at[b]).wait()
            pltpu.sync_copy(gbuf.at[b], agg_sh.at[dstv.at[g]], add=True)

    plsc.subcore_barrier()
    pltpu.sync_copy(agg_sh.at[pl.ds(sid * rows_per_tile, rows_per_tile)],
                    out_hbm.at[cid, pl.ds(sid * rows_per_tile, rows_per_tile)])


def _agg(x, src3d, dst3d):
    nwin = src3d.shape[1] * src3d.shape[2]
    body = functools.partial(_agg_body, nwin)
    return pl.kernel(
        body,
        out_type=jax.ShapeDtypeStruct((NC, NP, D), jnp.float32),
        mesh=_MESH,
        compiler_params=pltpu.CompilerParams(needs_layout_passes=False),
        scratch_types=[
            pltpu.VMEM_SHARED((NP, D), jnp.float32),
            pltpu.VMEM((2, AW, D), jnp.float32),
            pltpu.VMEM((ACHUNK, AW), jnp.int32),
            pltpu.VMEM((ACHUNK, AW), jnp.int32),
            pltpu.SemaphoreType.DMA((2,)),
        ],
    )(x, src3d, dst3d)


def _mlp_kernel(x_ref, agg_ref, wa_ref, ba_ref, wb_ref, bb_ref, out_ref):
    h = x_ref[...] + agg_ref[0] + agg_ref[1]
    w = jnp.dot(wa_ref[...], wb_ref[...], preferred_element_type=jnp.float32)
    b = jnp.dot(ba_ref[...], wb_ref[...], preferred_element_type=jnp.float32) + bb_ref[...]
    out_ref[...] = jnp.dot(h, w, preferred_element_type=jnp.float32) + b


def _mlp(x, agg, wa, ba, wb, bb):
    n, d = x.shape
    dout = wb.shape[1]
    blk = 1024
    return pl.pallas_call(
        _mlp_kernel,
        grid=(n // blk,),
        in_specs=[
            pl.BlockSpec((blk, d), lambda i: (i, 0)),
            pl.BlockSpec((NC, blk, d), lambda i: (0, i, 0)),
            pl.BlockSpec(wa.shape, lambda i: (0, 0)),
            pl.BlockSpec((1, ba.shape[0]), lambda i: (0, 0)),
            pl.BlockSpec(wb.shape, lambda i: (0, 0)),
            pl.BlockSpec((1, dout), lambda i: (0, 0)),
        ],
        out_specs=pl.BlockSpec((blk, dout), lambda i: (i, 0)),
        out_shape=jax.ShapeDtypeStruct((n, dout), jnp.float32),
    )(x, agg, wa, ba[None, :], wb, bb[None, :])


SCHUNK = 25


def _score_body(nwin, h_hbm, ii_hbm, jj_hbm, out_hbm,
                iv, jv, xi, xj, outv, sem):
    cid = lax.axis_index("c")
    sid = lax.axis_index("s")
    wid = cid * NS + sid

    lanes = lax.iota(jnp.int32, L)

    def _gather(g, b):
        pltpu.async_copy(h_hbm.at[iv.at[g]], xi.at[b], sem.at[2 * b])
        pltpu.async_copy(h_hbm.at[jv.at[g]], xj.at[b], sem.at[2 * b + 1])

    def _wait(g, b):
        pltpu.make_async_copy(h_hbm.at[iv.at[g]], xi.at[b], sem.at[2 * b]).wait()
        pltpu.make_async_copy(h_hbm.at[jv.at[g]], xj.at[b], sem.at[2 * b + 1]).wait()

    nwv = nwin
    @pl.loop(0, nwin)
    def _(gw):
        chunk = gw // SCHUNK
        g = lax.rem(gw, SCHUNK)
        b = lax.rem(gw, 2)

        @pl.when(g == 0)
        def _():
            pltpu.sync_copy(ii_hbm.at[wid, chunk], iv)
            pltpu.sync_copy(jj_hbm.at[wid, chunk], jv)
            _gather(0, b)

        @pl.when(jnp.logical_and(g + 1 < SCHUNK, gw + 1 < nwin))
        def _():
            _gather(g + 1, 1 - b)

        _wait(g, b)

        for grp in range(W // L):
            out_vec = jnp.zeros((L,), jnp.float32)
            for e16 in range(L):
                e = grp * L + e16
                p = xi[b, e, pl.ds(0, L)] * xj[b, e, pl.ds(0, L)]
                for k in range(1, D2 // L):
                    p = p + (xi[b, e, pl.ds(k * L, L)]
                             * xj[b, e, pl.ds(k * L, L)])
                out_vec = jnp.where(lanes == e16, jnp.sum(p), out_vec)
            outv[pl.ds(gw * W + grp * L, L)] = out_vec


def _score(h2, ii3d, jj3d):
    nwin = ii3d.shape[1] * ii3d.shape[2]
    e2 = NC * NS * nwin * W
    body = functools.partial(_score_body, nwin)
    return pl.kernel(
        body,
        out_type=jax.ShapeDtypeStruct((e2,), jnp.float32),
        mesh=_MESH,
        compiler_params=pltpu.CompilerParams(needs_layout_passes=False),
        scratch_types=[
            pltpu.VMEM((SCHUNK, W), jnp.int32),
            pltpu.VMEM((SCHUNK, W), jnp.int32),
            pltpu.VMEM((2, W, D), jnp.float32),
            pltpu.VMEM((2, W, D), jnp.float32),
            pltpu.VMEM((250 * W,), jnp.float32),
            pltpu.SemaphoreType.DMA((4,)),
        ],
    )(h2, ii3d, jj3d)


def kernel(x, W1a, b1a, W1b, b1b, W2a, b2a, W2b, b2b,
           train_pos_edge_index, pos_edge_index, neg_edge_index):
    nt = NC * NS
    xp = jnp.pad(x, ((0, NP - N), (0, 0)))
    src3d = train_pos_edge_index[0].reshape(nt, -1, ACHUNK, AW)
    dst3d = train_pos_edge_index[1].reshape(nt, -1, ACHUNK, AW)

    agg1 = _agg(xp, src3d, dst3d)
    h = _mlp(xp, agg1, W1a, b1a, W1b, b1b)
    agg2 = _agg(h, src3d, dst3d)
    W2bp = jnp.pad(W2b, ((0, 0), (0, D - D2)))
    b2bp = jnp.pad(b2b, (0, D - D2))
    h2 = _mlp(h, agg2, W2a, b2a, W2bp, b2bp)

    ii3d = jnp.concatenate([pos_edge_index[0], neg_edge_index[0]]).reshape(nt, -1, SCHUNK, W)
    jj3d = jnp.concatenate([pos_edge_index[1], neg_edge_index[1]]).reshape(nt, -1, SCHUNK, W)
    return _score(h2, ii3d, jj3d)
